# Initial kernel scaffold; baseline (speedup 1.0000x reference)
import jax
import jax.numpy as jnp
from jax.experimental import pallas as pl


def kernel(docs, thetas, emb, W, b):
    raise NotImplementedError("write your pallas kernel here")


# speedup vs baseline: 3.0363x; 3.0363x over previous
import functools

import jax
import jax.numpy as jnp
from jax import lax
from jax.experimental import pallas as pl
from jax.experimental.pallas import tpu as pltpu
from jax.experimental.pallas import tpu_sc as plsc

VOCAB = 1000000
DIM = 64
NDOCS = 16384
L = 200

NC = 2
NS = 16
NW = NC * NS
DPW = NDOCS // NW
CH = 16
NCHUNK = DPW // CH

VB = 20096
GRID = 50
VPAD = VB * GRID


def _matvec_body(emb_ref, w_ref, out_ref):
    i = pl.program_id(0)
    out_ref[pl.ds(i * VB, VB)] = jnp.sum(emb_ref[...] * w_ref[...], axis=1)


def _scores_tc(emb, w_row):
    return pl.pallas_call(
        _matvec_body,
        grid=(GRID,),
        in_specs=[
            pl.BlockSpec((VB, DIM), lambda i: (i, 0)),
            pl.BlockSpec((1, DIM), lambda i: (0, 0)),
        ],
        out_specs=pl.BlockSpec((VPAD,), lambda i: (0,)),
        out_shape=jax.ShapeDtypeStruct((VPAD,), jnp.float32),
    )(emb, w_row)


def _pool_sc(scores, docs_r, bias16):
    mesh = plsc.VectorSubcoreMesh(core_axis_name="c", subcore_axis_name="s")

    @functools.partial(
        pl.kernel,
        mesh=mesh,
        out_type=jax.ShapeDtypeStruct((NDOCS,), jnp.float32),
        scratch_types=[
            pltpu.VMEM((L * CH,), jnp.int32),
            pltpu.VMEM((L * CH,), jnp.float32),
            pltpu.VMEM((DPW,), jnp.float32),
            pltpu.VMEM((16,), jnp.float32),
            pltpu.SemaphoreType.DMA,
        ],
    )
    def k(scores_hbm, docs_hbm, bias_hbm, out_hbm, idx_v, vals_v, res_v,
          bias_v, sem):
        wid = lax.axis_index("s") * NC + lax.axis_index("c")
        pltpu.sync_copy(bias_hbm, bias_v)
        bvec = bias_v[...]

        def chunk_body(ci, _):
            flat = (wid * NCHUNK + ci) * (L * CH)
            pltpu.sync_copy(docs_hbm.at[pl.ds(flat, L * CH)], idx_v)
            pltpu.async_copy(scores_hbm.at[idx_v], vals_v, sem).wait()

            def red(l, acc):
                return acc + vals_v[pl.ds(l * CH, CH)]

            acc = lax.fori_loop(0, L, red, jnp.zeros((CH,), jnp.float32))
            x = acc * (1.0 / L) + bvec
            res_v[pl.ds(ci * CH, CH)] = 1.0 / (1.0 + jnp.exp(-x))
            return 0

        lax.fori_loop(0, NCHUNK, chunk_body, 0)
        pltpu.sync_copy(res_v, out_hbm.at[pl.ds(wid * DPW, DPW)])

    return k(scores, docs_r, bias16)


def kernel(docs, thetas, emb, W, b):
    del thetas
    docs_r = (
        docs.reshape(NW, NCHUNK, CH, L).transpose(0, 1, 3, 2).reshape(-1)
    )
    w_row = W.reshape(1, DIM)
    bias16 = jnp.broadcast_to(b, (16,)).astype(jnp.float32)
    scores = _scores_tc(emb, w_row)
    return _pool_sc(scores, docs_r, bias16)


# speedup vs baseline: 3.4179x; 1.1257x over previous
import functools

import jax
import jax.numpy as jnp
from jax import lax
from jax.experimental import pallas as pl
from jax.experimental.pallas import tpu as pltpu
from jax.experimental.pallas import tpu_sc as plsc

VOCAB = 1000000
DIM = 64
NDOCS = 16384
L = 200

NC = 2
NS = 16
NW = NC * NS
DPW = NDOCS // NW
CH = 16
NCHUNK = DPW // CH

VB = 25600
GRID = 40
VPAD = VB * GRID
KB = DIM * 128
QB = VB // 128
EB = VB * DIM // 128


def _matvec_body(emb_ref, bd_ref, out_ref):
    a = emb_ref[...].reshape(QB, KB)
    out_ref[...] = jax.lax.dot_general(
        a, bd_ref[...], (((1,), (0,)), ((), ())),
        preferred_element_type=jnp.float32)


def _scores_tc(emb_flat2d, bdiag):
    out2d = pl.pallas_call(
        _matvec_body,
        grid=(GRID,),
        in_specs=[
            pl.BlockSpec((EB, 128), lambda i: (i, 0)),
            pl.BlockSpec((KB, 128), lambda i: (0, 0)),
        ],
        out_specs=pl.BlockSpec((QB, 128), lambda i: (i, 0)),
        out_shape=jax.ShapeDtypeStruct((VPAD // 128, 128), jnp.float32),
    )(emb_flat2d, bdiag)
    return out2d.reshape(-1)


def _pool_sc(scores, docs_flat, bias16):
    mesh = plsc.VectorSubcoreMesh(core_axis_name="c", subcore_axis_name="s")

    @functools.partial(
        pl.kernel,
        mesh=mesh,
        out_type=jax.ShapeDtypeStruct((NDOCS,), jnp.float32),
        scratch_types=[
            pltpu.VMEM((L * CH,), jnp.int32),
            pltpu.VMEM((L * CH,), jnp.float32),
            pltpu.VMEM((DPW,), jnp.float32),
            pltpu.VMEM((16,), jnp.float32),
            pltpu.SemaphoreType.DMA,
        ],
        compiler_params=pltpu.CompilerParams(needs_layout_passes=False),
    )
    def k(scores_hbm, docs_hbm, bias_hbm, out_hbm, idx_v, vals_v, res_v,
          bias_v, sem):
        wid = lax.axis_index("s") * NC + lax.axis_index("c")
        pltpu.sync_copy(bias_hbm, bias_v)
        bvec = bias_v[...]
        lane_doc = lax.iota(jnp.int32, 16) * L

        def chunk_body(ci, _):
            flat = (wid * NCHUNK + ci) * (L * CH)
            pltpu.sync_copy(docs_hbm.at[pl.ds(flat, L * CH)], idx_v)
            pltpu.async_copy(scores_hbm.at[idx_v], vals_v, sem).wait()

            def red(l, acc):
                return acc + plsc.load_gather(vals_v, [lane_doc + l])

            acc = lax.fori_loop(0, L, red, jnp.zeros((CH,), jnp.float32))
            x = acc * (1.0 / L) + bvec
            res_v[pl.ds(ci * CH, CH)] = 1.0 / (1.0 + jnp.exp(-x))
            return 0

        lax.fori_loop(0, NCHUNK, chunk_body, 0)
        pltpu.sync_copy(res_v, out_hbm.at[pl.ds(wid * DPW, DPW)])

    return k(scores, docs_flat, bias16)


def kernel(docs, thetas, emb, W, b):
    del thetas
    emb_flat2d = emb.reshape(-1, 128)
    bdiag = jnp.kron(jnp.eye(128, dtype=jnp.float32), W.astype(jnp.float32))
    bias16 = jnp.broadcast_to(b, (16,)).astype(jnp.float32)
    scores = _scores_tc(emb_flat2d, bdiag)
    return _pool_sc(scores, docs.reshape(-1), bias16)
